# initial kernel scaffold (unmeasured)
import jax
import jax.numpy as jnp
from jax import lax
from jax.experimental import pallas as pl
from jax.experimental.pallas import tpu as pltpu


def kernel(
    x,
):
    def body(*refs):
        pass

    out_shape = jax.ShapeDtypeStruct(..., jnp.float32)
    return pl.pallas_call(body, out_shape=out_shape)(...)


# baseline (device time: 13370 ns/iter reference)
import jax
import jax.numpy as jnp
from jax import lax
from jax.experimental import pallas as pl
from jax.experimental.pallas import tpu as pltpu

N_DEV = 4
OUT_DTYPE = jnp.bfloat16


def kernel(x):
    m, n = x.shape
    sub = m // 128

    def body(x_ref, out_ref, send_buf, rstats, send_sems, recv_sems):
        my = lax.axis_index("i")

        barrier_sem = pltpu.get_barrier_semaphore()
        for k in range(1, N_DEV):
            pl.semaphore_signal(
                barrier_sem,
                inc=1,
                device_id=(lax.rem(my + k, N_DEV),),
                device_id_type=pl.DeviceIdType.MESH,
            )
        pl.semaphore_wait(barrier_sem, N_DEV - 1)

        xv = x_ref[:, :].astype(jnp.float32)
        mloc = jnp.max(xv, axis=1, keepdims=True)
        e = jnp.exp(xv - mloc)
        sloc = jnp.sum(e, axis=1, keepdims=True)

        ri = lax.broadcasted_iota(jnp.int32, (128, 128), 0)
        ci = lax.broadcasted_iota(jnp.int32, (128, 128), 1)
        eye = jnp.where(ri == ci, 1.0, 0.0).astype(jnp.float32)
        ones_r = jnp.ones((1, 128), jnp.float32)
        ones_c = jnp.ones((128, 1), jnp.float32)

        def col_to_row(col):
            return jnp.dot(ones_r, eye * col, preferred_element_type=jnp.float32)

        def row_to_col(row):
            return jnp.dot(eye * row, ones_c, preferred_element_type=jnp.float32)

        m_rows = jnp.concatenate(
            [col_to_row(mloc[b * 128 : (b + 1) * 128, :]) for b in range(sub)],
            axis=0,
        )
        s_rows = jnp.concatenate(
            [col_to_row(sloc[b * 128 : (b + 1) * 128, :]) for b in range(sub)],
            axis=0,
        )
        send_buf[0:sub, :] = m_rows
        send_buf[sub : 2 * sub, :] = s_rows

        out_ref[:, :] = e.astype(OUT_DTYPE)

        rdmas = []
        for k in range(1, N_DEV):
            rdma = pltpu.make_async_remote_copy(
                src_ref=send_buf,
                dst_ref=rstats.at[N_DEV - 1 - k],
                send_sem=send_sems.at[k - 1],
                recv_sem=recv_sems.at[N_DEV - 1 - k],
                device_id=(lax.rem(my + k, N_DEV),),
                device_id_type=pl.DeviceIdType.MESH,
            )
            rdma.start()
            rdmas.append(rdma)
        for rdma in rdmas:
            rdma.wait_send()
        for rdma in rdmas:
            rdma.wait_recv()

        m0 = send_buf[0:sub, :]
        s0 = send_buf[sub : 2 * sub, :]
        ms = [m0] + [rstats[j, 0:sub, :] for j in range(N_DEV - 1)]
        ss = [s0] + [rstats[j, sub : 2 * sub, :] for j in range(N_DEV - 1)]
        gmax = ms[0]
        for t in ms[1:]:
            gmax = jnp.maximum(gmax, t)
        gsum = ss[0] * jnp.exp(ms[0] - gmax)
        for tm, ts in zip(ms[1:], ss[1:]):
            gsum = gsum + ts * jnp.exp(tm - gmax)
        scale_rs = jnp.exp(m0 - gmax) / gsum

        scale = jnp.concatenate(
            [row_to_col(scale_rs[b : b + 1, :]) for b in range(sub)], axis=0
        )

        out_ref[:, :] = (out_ref[:, :].astype(jnp.float32) * scale).astype(
            OUT_DTYPE
        )

    return pl.pallas_call(
        body,
        out_shape=jax.ShapeDtypeStruct((m, n), OUT_DTYPE),
        in_specs=[pl.BlockSpec(memory_space=pltpu.VMEM)],
        out_specs=pl.BlockSpec(memory_space=pltpu.VMEM),
        scratch_shapes=[
            pltpu.VMEM((2 * sub, 128), jnp.float32),
            pltpu.VMEM((N_DEV - 1, 2 * sub, 128), jnp.float32),
            pltpu.SemaphoreType.DMA((N_DEV - 1,)),
            pltpu.SemaphoreType.DMA((N_DEV - 1,)),
        ],
        compiler_params=pltpu.CompilerParams(collective_id=0),
    )(x)


# device time: 12752 ns/iter; 1.0485x vs baseline; 1.0485x over previous
import jax
import jax.numpy as jnp
from jax import lax
from jax.experimental import pallas as pl
from jax.experimental.pallas import tpu as pltpu

N_DEV = 4
OUT_DTYPE = jnp.bfloat16


def kernel(x):
    m, n = x.shape
    sub = m // 128

    def body(x_ref, out_ref, send_buf, rstats, send_sems, recv_sems):
        my = lax.axis_index("i")

        barrier_sem = pltpu.get_barrier_semaphore()
        for k in range(1, N_DEV):
            pl.semaphore_signal(
                barrier_sem,
                inc=1,
                device_id=(lax.rem(my + k, N_DEV),),
                device_id_type=pl.DeviceIdType.MESH,
            )

        xv = x_ref[:, :].astype(jnp.float32)
        mloc = jnp.max(xv, axis=1, keepdims=True)
        e = jnp.exp(xv - mloc)
        sloc = jnp.sum(e, axis=1, keepdims=True)

        ri = lax.broadcasted_iota(jnp.int32, (128, 128), 0)
        ci = lax.broadcasted_iota(jnp.int32, (128, 128), 1)
        eye = jnp.where(ri == ci, 1.0, 0.0).astype(jnp.float32)
        ones_r = jnp.ones((1, 128), jnp.float32)
        ones_c = jnp.ones((128, 1), jnp.float32)

        def col_to_row(col):
            return jnp.dot(ones_r, eye * col, preferred_element_type=jnp.float32)

        def row_to_col(row):
            return jnp.dot(eye * row, ones_c, preferred_element_type=jnp.float32)

        m_rows = jnp.concatenate(
            [col_to_row(mloc[b * 128 : (b + 1) * 128, :]) for b in range(sub)],
            axis=0,
        )
        s_rows = jnp.concatenate(
            [col_to_row(sloc[b * 128 : (b + 1) * 128, :]) for b in range(sub)],
            axis=0,
        )
        send_buf[0:sub, :] = m_rows
        send_buf[sub : 2 * sub, :] = s_rows

        pl.semaphore_wait(barrier_sem, N_DEV - 1)

        rdmas = []
        for k in range(1, N_DEV):
            rdma = pltpu.make_async_remote_copy(
                src_ref=send_buf,
                dst_ref=rstats.at[N_DEV - 1 - k],
                send_sem=send_sems.at[k - 1],
                recv_sem=recv_sems.at[N_DEV - 1 - k],
                device_id=(lax.rem(my + k, N_DEV),),
                device_id_type=pl.DeviceIdType.MESH,
            )
            rdma.start()
            rdmas.append(rdma)

        out_ref[:, :] = e.astype(OUT_DTYPE)

        for rdma in rdmas:
            rdma.wait_recv()
        for rdma in rdmas:
            rdma.wait_send()

        m0 = send_buf[0:sub, :]
        s0 = send_buf[sub : 2 * sub, :]
        ms = [m0] + [rstats[j, 0:sub, :] for j in range(N_DEV - 1)]
        ss = [s0] + [rstats[j, sub : 2 * sub, :] for j in range(N_DEV - 1)]
        gmax = ms[0]
        for t in ms[1:]:
            gmax = jnp.maximum(gmax, t)
        gsum = ss[0] * jnp.exp(ms[0] - gmax)
        for tm, ts in zip(ms[1:], ss[1:]):
            gsum = gsum + ts * jnp.exp(tm - gmax)
        scale_rs = jnp.exp(m0 - gmax) / gsum

        scale = jnp.concatenate(
            [row_to_col(scale_rs[b : b + 1, :]) for b in range(sub)], axis=0
        )

        out_ref[:, :] = (out_ref[:, :].astype(jnp.float32) * scale).astype(
            OUT_DTYPE
        )

    return pl.pallas_call(
        body,
        out_shape=jax.ShapeDtypeStruct((m, n), OUT_DTYPE),
        in_specs=[pl.BlockSpec(memory_space=pltpu.VMEM)],
        out_specs=pl.BlockSpec(memory_space=pltpu.VMEM),
        scratch_shapes=[
            pltpu.VMEM((2 * sub, 128), jnp.float32),
            pltpu.VMEM((N_DEV - 1, 2 * sub, 128), jnp.float32),
            pltpu.SemaphoreType.DMA((N_DEV - 1,)),
            pltpu.SemaphoreType.DMA((N_DEV - 1,)),
        ],
        compiler_params=pltpu.CompilerParams(collective_id=0),
    )(x)
